# Initial kernel scaffold; baseline (speedup 1.0000x reference)
import jax
import jax.numpy as jnp
from jax.experimental import pallas as pl


def kernel(conf_preds, loc_preds, conf_targets, loc_targets):
    raise NotImplementedError("write your pallas kernel here")


# speedup vs baseline: 1.2533x; 1.2533x over previous
import functools

import jax
import jax.numpy as jnp
from jax.experimental import pallas as pl
from jax.experimental.pallas import tpu as pltpu

_ROWS = 4096


def _phase1_body(n_total, conf_ref, tgt_ref, locp_ref, loct_ref,
                 nll_ref, locacc_ref):
    i = pl.program_id(0)
    x = conf_ref[...]
    t = jnp.maximum(tgt_ref[...], 0)
    m = jnp.max(x, axis=1, keepdims=True)
    s = jnp.sum(jnp.exp(x - m), axis=1, keepdims=True)
    lse = jnp.log(s) + m
    cids = jax.lax.broadcasted_iota(jnp.int32, x.shape, 1)
    gathered = jnp.sum(jnp.where(cids == t, x, 0.0), axis=1, keepdims=True)
    nll_ref[...] = lse - gathered

    dp = locp_ref[...] - loct_ref[...]
    ad = jnp.abs(dp)
    sl1 = jnp.where(ad < 1.0, 0.5 * dp * dp, ad - 0.5)
    rows = jax.lax.broadcasted_iota(jnp.int32, (x.shape[0], 1), 0) + i * _ROWS
    valid = jnp.logical_and(rows < n_total, t > 0)
    contrib = jnp.sum(
        jnp.where(valid, jnp.sum(sl1, axis=1, keepdims=True), 0.0),
        keepdims=True)

    @pl.when(i == 0)
    def _():
        locacc_ref[...] = contrib

    @pl.when(i > 0)
    def _():
        locacc_ref[...] += contrib


def _phase2_body(a_dim, nll_ref, tgt_ref, locacc_ref,
                 conf_ref, loc_ref, topk_scr):
    nll = nll_ref[...]
    t = jnp.maximum(tgt_ref[...], 0)
    pos = t > 0
    posf = pos.astype(jnp.float32)
    num_pos = jnp.sum(posf, axis=1, keepdims=True)
    num_pos_i = num_pos.astype(jnp.int32)
    k = jnp.minimum(3 * num_pos_i, a_dim - 1)
    n_neg = a_dim - num_pos_i
    sum_pos = jnp.sum(nll * posf, axis=1, keepdims=True)
    negv = jnp.where(pos, 0.0, nll)
    sum_allneg = jnp.sum(negv, axis=1, keepdims=True)
    full = k >= n_neg
    needs = jnp.logical_and(jnp.logical_not(full), k > 0)

    topk_scr[...] = jnp.zeros_like(topk_scr)

    @pl.when(jnp.any(needs))
    def _():
        def body(_, carry):
            lo, hi = carry
            mid = lo + (hi - lo + 1) // 2
            thr = jax.lax.bitcast_convert_type(mid, jnp.float32)
            cnt = jnp.sum((negv >= thr).astype(jnp.float32),
                          axis=1, keepdims=True).astype(jnp.int32)
            ge = cnt >= k
            return jnp.where(ge, mid, lo), jnp.where(ge, hi, mid - 1)

        lo0 = jnp.zeros_like(k)
        hi0 = jnp.full_like(k, 0x7F800000)
        lo, _ = jax.lax.fori_loop(0, 31, body, (lo0, hi0))
        vk = jax.lax.bitcast_convert_type(lo, jnp.float32)
        gt = (negv > vk).astype(jnp.float32)
        n_gt = jnp.sum(gt, axis=1, keepdims=True)
        s_gt = jnp.sum(negv * gt, axis=1, keepdims=True)
        topk = s_gt + (k.astype(jnp.float32) - n_gt) * vk
        topk_scr[...] = jnp.where(needs, topk, 0.0)

    conf_rows = sum_pos + jnp.where(full, sum_allneg, topk_scr[...])
    n_tot = jnp.maximum(jnp.sum(num_pos, keepdims=True), 1.0)
    conf_ref[...] = jnp.sum(conf_rows, keepdims=True) / n_tot
    loc_ref[...] = locacc_ref[...] / n_tot


def kernel(conf_preds, loc_preds, conf_targets, loc_targets):
    B, A, C = conf_preds.shape
    N = B * A
    conf2 = conf_preds.reshape(N, C)
    tgt2 = conf_targets.reshape(N, 1).astype(jnp.int32)
    locp2 = loc_preds.reshape(N, 4)
    loct2 = loc_targets.reshape(N, 4)

    grid = (pl.cdiv(N, _ROWS),)
    nll, locacc = pl.pallas_call(
        functools.partial(_phase1_body, N),
        grid=grid,
        in_specs=[
            pl.BlockSpec((_ROWS, C), lambda i: (i, 0)),
            pl.BlockSpec((_ROWS, 1), lambda i: (i, 0)),
            pl.BlockSpec((_ROWS, 4), lambda i: (i, 0)),
            pl.BlockSpec((_ROWS, 4), lambda i: (i, 0)),
        ],
        out_specs=[
            pl.BlockSpec((_ROWS, 1), lambda i: (i, 0)),
            pl.BlockSpec((1, 1), lambda i: (0, 0)),
        ],
        out_shape=[
            jax.ShapeDtypeStruct((N, 1), jnp.float32),
            jax.ShapeDtypeStruct((1, 1), jnp.float32),
        ],
    )(conf2, tgt2, locp2, loct2)

    conf_out, loc_out = pl.pallas_call(
        functools.partial(_phase2_body, A),
        in_specs=[
            pl.BlockSpec((B, A), lambda: (0, 0)),
            pl.BlockSpec((B, A), lambda: (0, 0)),
            pl.BlockSpec((1, 1), lambda: (0, 0)),
        ],
        out_specs=[
            pl.BlockSpec((1, 1), lambda: (0, 0)),
            pl.BlockSpec((1, 1), lambda: (0, 0)),
        ],
        out_shape=[
            jax.ShapeDtypeStruct((1, 1), jnp.float32),
            jax.ShapeDtypeStruct((1, 1), jnp.float32),
        ],
        scratch_shapes=[pltpu.VMEM((B, 1), jnp.float32)],
    )(nll.reshape(B, A), conf_targets.astype(jnp.int32), locacc)

    return conf_out[0, 0], loc_out[0, 0]
